# baseline (speedup 1.0000x reference)
import jax
import jax.numpy as jnp
from jax import lax
from jax.experimental import pallas as pl
from jax.experimental.pallas import tpu as pltpu
from jax.experimental.pallas import tpu_sc as plsc

F = 32
B = 16384
NC, NS, L = 2, 16, 16
NW = NC * NS
BPW = B // NW
CH = 4
NCH = BPW // CH
NPAIR = NCH // 2


def _body(uij_hbm, xT_hbm, yT_hbm, out_hbm,
          idx_v, bx0, by0, bz0, bx1, by1, bz1, out_v, sem0, sem1):
    wid = lax.axis_index("s") * NC + lax.axis_index("c")
    lane = lax.iota(jnp.int32, L)
    fidx0 = lax.iota(jnp.int32, L)
    fidx1 = fidx0 + L
    pltpu.sync_copy(uij_hbm.at[wid], idx_v)

    def copies(c, bufs, sem):
        bufx, bufy, bufz = bufs
        iv = idx_v[c, pl.ds(0, L)]
        out = []
        for k in range(CH):
            au = pl.multiple_of((iv[k] >> 7) * 128, 128)
            ai = pl.multiple_of((iv[CH + k] >> 7) * 128, 128)
            aj = pl.multiple_of((iv[2 * CH + k] >> 7) * 128, 128)
            out.append(pltpu.make_async_copy(
                xT_hbm.at[:, pl.ds(au, 128)], bufx.at[k], sem))
            out.append(pltpu.make_async_copy(
                yT_hbm.at[:, pl.ds(ai, 128)], bufy.at[k], sem))
            out.append(pltpu.make_async_copy(
                yT_hbm.at[:, pl.ds(aj, 128)], bufz.at[k], sem))
        return out

    def compute(c, bufs, acc):
        bufx, bufy, bufz = bufs
        iv = idx_v[c, pl.ds(0, L)]
        for k in range(CH):
            lu = jnp.full((L,), iv[k] & 127, jnp.int32)
            li = jnp.full((L,), iv[CH + k] & 127, jnp.int32)
            lj = jnp.full((L,), iv[2 * CH + k] & 127, jnp.int32)
            bvec = jnp.full((L,), k, jnp.int32)
            p = jnp.zeros((L,), jnp.float32)
            for fidx in (fidx0, fidx1):
                xu = plsc.load_gather(bufx, [bvec, fidx, lu])
                yi = plsc.load_gather(bufy, [bvec, fidx, li])
                yj = plsc.load_gather(bufz, [bvec, fidx, lj])
                p = p + xu * (yi - yj)
            s = jnp.sum(p)
            acc = jnp.where(lane == (c % 4) * CH + k, s, acc)
        return acc

    set0 = (bx0, by0, bz0)
    set1 = (bx1, by1, bz1)

    for cp in copies(0, set0, sem0):
        cp.start()

    def pair(cc, acc):
        c0 = 2 * cc
        c1 = c0 + 1
        for cp in copies(c1, set1, sem1):
            cp.start()
        for cp in copies(c0, set0, sem0):
            cp.wait()
        acc = compute(c0, set0, acc)

        @pl.when(cc < NPAIR - 1)
        def _():
            for cp in copies(c0 + 2, set0, sem0):
                cp.start()

        for cp in copies(c1, set1, sem1):
            cp.wait()
        acc = compute(c1, set1, acc)

        @pl.when(cc % 2 == 1)
        def _():
            out_v[pl.ds((cc // 2) * L, L)] = acc

        return acc

    lax.fori_loop(0, NPAIR, pair, jnp.zeros((L,), jnp.float32))
    pltpu.sync_copy(out_v, out_hbm.at[pl.ds(wid * BPW, BPW)])


def kernel(user, item_i, item_j, x, y):
    mesh = plsc.VectorSubcoreMesh(core_axis_name="c", subcore_axis_name="s",
                                  num_cores=NC, num_subcores=NS)
    run = pl.kernel(
        _body,
        out_type=jax.ShapeDtypeStruct((B,), jnp.float32),
        mesh=mesh,
        compiler_params=pltpu.CompilerParams(needs_layout_passes=False,
                                             use_tc_tiling_on_sc=True),
        scratch_types=[
            pltpu.VMEM((NCH, L), jnp.int32),
            pltpu.VMEM((CH, F, 128), jnp.float32),
            pltpu.VMEM((CH, F, 128), jnp.float32),
            pltpu.VMEM((CH, F, 128), jnp.float32),
            pltpu.VMEM((CH, F, 128), jnp.float32),
            pltpu.VMEM((CH, F, 128), jnp.float32),
            pltpu.VMEM((CH, F, 128), jnp.float32),
            pltpu.VMEM((BPW,), jnp.float32),
            pltpu.SemaphoreType.DMA,
            pltpu.SemaphoreType.DMA,
        ],
    )
    idx = jnp.stack([user.astype(jnp.int32),
                     item_i.astype(jnp.int32),
                     item_j.astype(jnp.int32)]).reshape(3, NW, NCH, CH)
    uij = jnp.concatenate(
        [jnp.transpose(idx, (1, 2, 0, 3)).reshape(NW, NCH, 3 * CH),
         jnp.zeros((NW, NCH, L - 3 * CH), jnp.int32)], axis=-1)
    return run(uij, x.T, y.T)


# speedup vs baseline: 1.0229x; 1.0229x over previous
import jax
import jax.numpy as jnp
from jax import lax
from jax.experimental import pallas as pl
from jax.experimental.pallas import tpu as pltpu
from jax.experimental.pallas import tpu_sc as plsc

F = 32
B = 16384
NC, NS, L = 2, 16, 16
NW = NC * NS
BPW = B // NW
CH = 8
NCH = BPW // CH


def _body(uij_hbm, xT_hbm, yT_hbm, out_hbm,
          idx_v, bufx, bufy, bufz, out_v, sem):
    wid = lax.axis_index("s") * NC + lax.axis_index("c")
    lane = lax.iota(jnp.int32, L)
    fidx0 = lax.iota(jnp.int32, L)
    fidx1 = fidx0 + L
    pltpu.sync_copy(uij_hbm.at[wid], idx_v)

    def chunk(c, acc):
        iv0 = idx_v[c, pl.ds(0, L)]
        iv1 = idx_v[c, pl.ds(CH, L)]

        def blocks(k):
            u = iv0[k]
            i_ = iv1[k]
            j_ = iv1[CH + k]
            au = pl.multiple_of((u >> 7) * 128, 128)
            ai = pl.multiple_of((i_ >> 7) * 128, 128)
            aj = pl.multiple_of((j_ >> 7) * 128, 128)
            return (
                pltpu.make_async_copy(xT_hbm.at[:, pl.ds(au, 128)], bufx.at[k], sem),
                pltpu.make_async_copy(yT_hbm.at[:, pl.ds(ai, 128)], bufy.at[k], sem),
                pltpu.make_async_copy(yT_hbm.at[:, pl.ds(aj, 128)], bufz.at[k], sem),
            )

        for k in range(CH):
            for cp in blocks(k):
                cp.start()
        for k in range(CH):
            for cp in blocks(k):
                cp.wait()

        for k in range(CH):
            lu = jnp.full((L,), iv0[k] & 127, jnp.int32)
            li = jnp.full((L,), iv1[k] & 127, jnp.int32)
            lj = jnp.full((L,), iv1[CH + k] & 127, jnp.int32)
            bvec = jnp.full((L,), k, jnp.int32)
            p = jnp.zeros((L,), jnp.float32)
            for fidx in (fidx0, fidx1):
                xu = plsc.load_gather(bufx, [bvec, fidx, lu])
                yi = plsc.load_gather(bufy, [bvec, fidx, li])
                yj = plsc.load_gather(bufz, [bvec, fidx, lj])
                p = p + xu * (yi - yj)
            s = jnp.sum(p)
            acc = jnp.where(lane == ((c % 2) * CH + k), s, acc)

        @pl.when(c % 2 == 1)
        def _():
            out_v[pl.ds((c // 2) * L, L)] = acc

        return acc

    lax.fori_loop(0, NCH, chunk, jnp.zeros((L,), jnp.float32))
    pltpu.sync_copy(out_v, out_hbm.at[pl.ds(wid * BPW, BPW)])


def kernel(user, item_i, item_j, x, y):
    mesh = plsc.VectorSubcoreMesh(core_axis_name="c", subcore_axis_name="s",
                                  num_cores=NC, num_subcores=NS)
    run = pl.kernel(
        _body,
        out_type=jax.ShapeDtypeStruct((B,), jnp.float32),
        mesh=mesh,
        compiler_params=pltpu.CompilerParams(needs_layout_passes=False,
                                             use_tc_tiling_on_sc=True),
        scratch_types=[
            pltpu.VMEM((NCH, 3 * CH), jnp.int32),
            pltpu.VMEM((CH, F, 128), jnp.float32),
            pltpu.VMEM((CH, F, 128), jnp.float32),
            pltpu.VMEM((CH, F, 128), jnp.float32),
            pltpu.VMEM((BPW,), jnp.float32),
            pltpu.SemaphoreType.DMA,
        ],
    )
    idx = jnp.stack([user.astype(jnp.int32),
                     item_i.astype(jnp.int32),
                     item_j.astype(jnp.int32)]).reshape(3, NW, NCH, CH)
    uij = jnp.transpose(idx, (1, 2, 0, 3)).reshape(NW, NCH, 3 * CH)
    return run(uij, x.T, y.T)
